# baseline (speedup 1.0000x reference)
import jax
import jax.numpy as jnp
from jax.experimental import pallas as pl
from jax.experimental.pallas import tpu as pltpu

_B = 1024
_CHUNK = 64
_NCODE = 256
_SOMX = 16
_SOMY = 16
_LAT = 64
_HP = jax.lax.Precision.HIGHEST


def _lrelu(x):
    return jnp.where(x > 0, x, 0.01 * x)


def _decode(z, wdt, wd0t, wd1t, wd2t):
    d = _lrelu(jnp.dot(z, wdt))
    d = _lrelu(jnp.dot(d, wd0t))
    d = _lrelu(jnp.dot(d, wd1t))
    d = _lrelu(jnp.dot(d, wd2t))
    return d


def _body(x_ref, eps_ref, emb_ref, w0_ref, w1t_ref, wmut_ref, wlvt_ref,
          wdt_ref, wd0t_ref, wd1t_ref, wd2t_ref,
          ze_ref, zq_ref, up_ref, dn_ref, lf_ref, de_ref, dq_ref):
    x = x_ref[...]
    h1 = _lrelu(x * w0_ref[...])
    h2 = _lrelu(jnp.dot(h1, w1t_ref[...]))
    mu = jnp.dot(h2, wmut_ref[...])
    lv = jnp.dot(h2, wlvt_ref[...])
    std = jnp.exp(0.5 * lv)
    z_e = mu + eps_ref[...] * std
    ze_ref[...] = z_e

    emb = emb_ref[...]
    diff = z_e[:, None, :] - emb[None, :, :]
    dist = jnp.sum(diff * diff, axis=-1)
    m = jnp.min(dist, axis=-1, keepdims=True)
    iota = jax.lax.broadcasted_iota(jnp.int32, dist.shape, 1)
    nmin = jnp.min(jnp.where(dist == m, iota, _NCODE), axis=-1)

    nx = nmin // _SOMY
    ny = nmin % _SOMY
    up_ok = nx < (_SOMX - 1)
    dn_ok = nx > 0
    lf_ok = ny > 0

    def onehot(idx, ok):
        oh = (iota == idx[:, None]) & ok[:, None]
        return oh.astype(jnp.float32)

    ones = jnp.ones(nmin.shape, dtype=jnp.bool_)
    z_q = jnp.dot(onehot(nmin, ones), emb, precision=_HP)
    zq_ref[...] = z_q
    up_ref[...] = jnp.dot(onehot(nmin + _SOMY, up_ok), emb, precision=_HP)
    dn_ref[...] = jnp.dot(onehot(nmin - _SOMY, dn_ok), emb, precision=_HP)
    lf_ref[...] = jnp.dot(onehot(nmin - 1, lf_ok), emb, precision=_HP)

    wdt, wd0t, wd1t, wd2t = wdt_ref[...], wd0t_ref[...], wd1t_ref[...], wd2t_ref[...]
    de_ref[...] = _decode(z_e, wdt, wd0t, wd1t, wd2t)
    dq_ref[...] = _decode(z_q, wdt, wd0t, wd1t, wd2t)


def kernel(x, eps, embeddings, W_enc0, b_enc0, W_enc1, b_enc1, W_mu, b_mu,
           W_lv, b_lv, W_dec, b_dec, W_dec0, b_dec0, W_dec1, b_dec1,
           W_dec2, b_dec2):
    del b_enc0, b_enc1, b_mu, b_lv, b_dec, b_dec0, b_dec1, b_dec2
    emb = embeddings.reshape(_NCODE, _LAT)
    w0 = W_enc0.reshape(1, 10)
    w1t = W_enc1.T
    wmut = W_mu.T
    wlvt = W_lv.T
    wdt = W_dec.T
    wd0t = W_dec0.T
    wd1t = W_dec1.T
    wd2t = W_dec2.T

    grid = (_B // _CHUNK,)

    def chunk_spec(ncol):
        return pl.BlockSpec((_CHUNK, ncol), lambda i: (i, 0))

    def const_spec(shape):
        return pl.BlockSpec(shape, lambda i: (0, 0))

    out_shapes = (
        jax.ShapeDtypeStruct((_B, _LAT), jnp.float32),
        jax.ShapeDtypeStruct((_B, _LAT), jnp.float32),
        jax.ShapeDtypeStruct((_B, _LAT), jnp.float32),
        jax.ShapeDtypeStruct((_B, _LAT), jnp.float32),
        jax.ShapeDtypeStruct((_B, _LAT), jnp.float32),
        jax.ShapeDtypeStruct((_B, 1), jnp.float32),
        jax.ShapeDtypeStruct((_B, 1), jnp.float32),
    )
    in_specs = [
        chunk_spec(1),
        chunk_spec(_LAT),
        const_spec((_NCODE, _LAT)),
        const_spec((1, 10)),
        const_spec((10, 50)),
        const_spec((50, _LAT)),
        const_spec((50, _LAT)),
        const_spec((_LAT, 100)),
        const_spec((100, 60)),
        const_spec((60, 30)),
        const_spec((30, 1)),
    ]
    out_specs = (
        chunk_spec(_LAT), chunk_spec(_LAT), chunk_spec(_LAT),
        chunk_spec(_LAT), chunk_spec(_LAT), chunk_spec(1), chunk_spec(1),
    )
    z_e, z_q, up, dn, lf, de, dq = pl.pallas_call(
        _body,
        grid=grid,
        in_specs=in_specs,
        out_specs=out_specs,
        out_shape=out_shapes,
    )(x, eps, emb, w0, w1t, wmut, wlvt, wdt, wd0t, wd1t, wd2t)

    zeros = jnp.zeros_like(z_q)
    z_q_neighbors = jnp.stack([z_q, up, dn, zeros, lf], axis=1)
    return (z_e, z_q, z_q_neighbors, de, dq)


# speedup vs baseline: 2.5712x; 2.5712x over previous
import jax
import jax.numpy as jnp
from jax.experimental import pallas as pl
from jax.experimental.pallas import tpu as pltpu

_B = 1024
_CHUNK = 128
_NCODE = 256
_SOMX = 16
_SOMY = 16
_LAT = 64
_NCAND = 4
_HP = jax.lax.Precision.HIGHEST


def _lrelu(x):
    return jnp.where(x > 0, x, 0.01 * x)


def _decode(z, wdt, wd0t, wd1t, wd2t):
    d = _lrelu(jnp.dot(z, wdt))
    d = _lrelu(jnp.dot(d, wd0t))
    d = _lrelu(jnp.dot(d, wd1t))
    d = _lrelu(jnp.dot(d, wd2t))
    return d


def _body(x_ref, eps_ref, emb_ref, embt_ref, w0_ref, w1t_ref, wmut_ref,
          wlvt_ref, wdt_ref, wd0t_ref, wd1t_ref, wd2t_ref,
          ze_ref, zq_ref, up_ref, dn_ref, lf_ref, de_ref, dq_ref):
    x = x_ref[...]
    h1 = _lrelu(x * w0_ref[...])
    h2 = _lrelu(jnp.dot(h1, w1t_ref[...]))
    mu = jnp.dot(h2, wmut_ref[...])
    lv = jnp.dot(h2, wlvt_ref[...])
    std = jnp.exp(0.5 * lv)
    z_e = mu + eps_ref[...] * std
    ze_ref[...] = z_e

    emb = emb_ref[...]
    embt = embt_ref[...]
    sumsq_e = jnp.sum(embt * embt, axis=0)
    scores = (sumsq_e[None, :]
              - 2.0 * jnp.dot(z_e, embt, precision=_HP))

    iota = jax.lax.broadcasted_iota(jnp.int32, scores.shape, 1)
    big = jnp.float32(jnp.inf)
    d_work = scores
    best_d = None
    best_i = None
    for _ in range(_NCAND):
        m = jnp.min(d_work, axis=-1, keepdims=True)
        idx_k = jnp.min(jnp.where(d_work == m, iota, _NCODE), axis=-1)
        oh_k = (iota == idx_k[:, None]).astype(jnp.float32)
        d_work = jnp.where(iota == idx_k[:, None], big, d_work)
        cand = jnp.dot(oh_k, emb, precision=_HP)
        diff = z_e - cand
        d_k = jnp.sum(diff * diff, axis=-1)
        if best_d is None:
            best_d, best_i = d_k, idx_k
        else:
            take = (d_k < best_d) | ((d_k == best_d) & (idx_k < best_i))
            best_d = jnp.where(take, d_k, best_d)
            best_i = jnp.where(take, idx_k, best_i)
    nmin = best_i

    nx = nmin // _SOMY
    ny = nmin % _SOMY
    up_ok = nx < (_SOMX - 1)
    dn_ok = nx > 0
    lf_ok = ny > 0

    def onehot(idx, ok):
        oh = (iota == idx[:, None]) & ok[:, None]
        return oh.astype(jnp.float32)

    ones = jnp.ones(nmin.shape, dtype=jnp.bool_)
    z_q = jnp.dot(onehot(nmin, ones), emb, precision=_HP)
    zq_ref[...] = z_q
    up_ref[...] = jnp.dot(onehot(nmin + _SOMY, up_ok), emb, precision=_HP)
    dn_ref[...] = jnp.dot(onehot(nmin - _SOMY, dn_ok), emb, precision=_HP)
    lf_ref[...] = jnp.dot(onehot(nmin - 1, lf_ok), emb, precision=_HP)

    wdt, wd0t, wd1t, wd2t = wdt_ref[...], wd0t_ref[...], wd1t_ref[...], wd2t_ref[...]
    de_ref[...] = _decode(z_e, wdt, wd0t, wd1t, wd2t)
    dq_ref[...] = _decode(z_q, wdt, wd0t, wd1t, wd2t)


def kernel(x, eps, embeddings, W_enc0, b_enc0, W_enc1, b_enc1, W_mu, b_mu,
           W_lv, b_lv, W_dec, b_dec, W_dec0, b_dec0, W_dec1, b_dec1,
           W_dec2, b_dec2):
    del b_enc0, b_enc1, b_mu, b_lv, b_dec, b_dec0, b_dec1, b_dec2
    emb = embeddings.reshape(_NCODE, _LAT)
    embt = emb.T
    w0 = W_enc0.reshape(1, 10)
    w1t = W_enc1.T
    wmut = W_mu.T
    wlvt = W_lv.T
    wdt = W_dec.T
    wd0t = W_dec0.T
    wd1t = W_dec1.T
    wd2t = W_dec2.T

    grid = (_B // _CHUNK,)

    def chunk_spec(ncol):
        return pl.BlockSpec((_CHUNK, ncol), lambda i: (i, 0))

    def const_spec(shape):
        return pl.BlockSpec(shape, lambda i: (0, 0))

    out_shapes = (
        jax.ShapeDtypeStruct((_B, _LAT), jnp.float32),
        jax.ShapeDtypeStruct((_B, _LAT), jnp.float32),
        jax.ShapeDtypeStruct((_B, _LAT), jnp.float32),
        jax.ShapeDtypeStruct((_B, _LAT), jnp.float32),
        jax.ShapeDtypeStruct((_B, _LAT), jnp.float32),
        jax.ShapeDtypeStruct((_B, 1), jnp.float32),
        jax.ShapeDtypeStruct((_B, 1), jnp.float32),
    )
    in_specs = [
        chunk_spec(1),
        chunk_spec(_LAT),
        const_spec((_NCODE, _LAT)),
        const_spec((_LAT, _NCODE)),
        const_spec((1, 10)),
        const_spec((10, 50)),
        const_spec((50, _LAT)),
        const_spec((50, _LAT)),
        const_spec((_LAT, 100)),
        const_spec((100, 60)),
        const_spec((60, 30)),
        const_spec((30, 1)),
    ]
    out_specs = (
        chunk_spec(_LAT), chunk_spec(_LAT), chunk_spec(_LAT),
        chunk_spec(_LAT), chunk_spec(_LAT), chunk_spec(1), chunk_spec(1),
    )
    z_e, z_q, up, dn, lf, de, dq = pl.pallas_call(
        _body,
        grid=grid,
        in_specs=in_specs,
        out_specs=out_specs,
        out_shape=out_shapes,
    )(x, eps, emb, embt, w0, w1t, wmut, wlvt, wdt, wd0t, wd1t, wd2t)

    zeros = jnp.zeros_like(z_q)
    z_q_neighbors = jnp.stack([z_q, up, dn, zeros, lf], axis=1)
    return (z_e, z_q, z_q_neighbors, de, dq)
